# Initial kernel scaffold; baseline (speedup 1.0000x reference)
import jax
import jax.numpy as jnp
from jax.experimental import pallas as pl


def kernel(x, positions_unmask, positions_mask, pos_table):
    raise NotImplementedError("write your pallas kernel here")


# speedup vs baseline: 1.8237x; 1.8237x over previous
import functools
import jax
import jax.numpy as jnp
from jax import lax
from jax.experimental import pallas as pl
from jax.experimental.pallas import tpu as pltpu
from jax.experimental.pallas import tpu_sc as plsc

_NW = 32
_CHUNK = 256
_J = _CHUNK // 128


def _make_sc_kernel(n_u, n_m, D, pu_w, pm_w):
    mesh = plsc.VectorSubcoreMesh(core_axis_name="c", subcore_axis_name="s")

    @functools.partial(
        pl.kernel,
        out_type=(
            jax.ShapeDtypeStruct((n_u, D + 1), jnp.float32),
            jax.ShapeDtypeStruct((n_m, D), jnp.float32),
        ),
        mesh=mesh,
        compiler_params=pltpu.CompilerParams(
            use_tc_tiling_on_sc=False, needs_layout_passes=False),
        scratch_types=[
            pltpu.VMEM((_J, 128), jnp.int32),
            pltpu.VMEM((_CHUNK, D + 1), jnp.float32),
            pltpu.VMEM((_CHUNK, D), jnp.float32),
            pltpu.VMEM((_CHUNK,), jnp.float32),
            pltpu.SemaphoreType.DMA,
        ],
    )
    def k(aug_hbm, tab_hbm, iu_hbm, im_hbm, x_hbm, out_hbm, pm_hbm,
          idx_v, rows_v, pmr_v, xc_v, sem):
        wid = lax.axis_index("s") * 2 + lax.axis_index("c")

        def u_body(j, carry):
            base = wid * pu_w + j * _CHUNK
            pltpu.sync_copy(iu_hbm.at[pl.ds(base // 128, _J)], idx_v)
            cps = [
                pltpu.async_copy(
                    aug_hbm.at[idx_v.at[t]],
                    rows_v.at[pl.ds(t * 128, 128)], sem)
                for t in range(_J)
            ]
            pltpu.sync_copy(x_hbm.at[pl.ds(base, _CHUNK)], xc_v)
            for cp in cps:
                cp.wait()
            zero16 = jnp.zeros((16,), jnp.int32)
            for t in range(_CHUNK // 16):
                vals = xc_v[pl.ds(t * 16, 16)]
                rows = lax.iota(jnp.int32, 16) + (t * 16)
                plsc.store_scatter(rows_v, [rows, zero16], vals)
            pltpu.sync_copy(rows_v, out_hbm.at[pl.ds(base, _CHUNK)])
            return carry

        lax.fori_loop(0, pu_w // _CHUNK, u_body, 0)

        def m_body(j, carry):
            base = wid * pm_w + j * _CHUNK
            pltpu.sync_copy(im_hbm.at[pl.ds(base // 128, _J)], idx_v)
            cps = [
                pltpu.async_copy(
                    tab_hbm.at[idx_v.at[t]],
                    pmr_v.at[pl.ds(t * 128, 128)], sem)
                for t in range(_J)
            ]
            for cp in cps:
                cp.wait()
            pltpu.sync_copy(pmr_v, pm_hbm.at[pl.ds(base, _CHUNK)])
            return carry

        lax.fori_loop(0, pm_w // _CHUNK, m_body, 0)

    return k


def kernel(x, positions_unmask, positions_mask, pos_table):
    B, L = x.shape
    LM = positions_mask.shape[1]
    D = pos_table.shape[1]
    n_u = B * L
    n_m = B * LM
    pu_w = n_u // _NW
    pm_w = n_m // _NW

    xf = x.astype(jnp.float32).reshape(n_u)
    iu = positions_unmask.astype(jnp.int32).reshape(n_u // 128, 128)
    im = positions_mask.astype(jnp.int32).reshape(n_m // 128, 128)
    aug = jnp.concatenate(
        [jnp.zeros((pos_table.shape[0], 1), jnp.float32), pos_table], axis=1)

    k = _make_sc_kernel(n_u, n_m, D, pu_w, pm_w)
    out, pm = k(aug, pos_table, iu, im, xf)
    return out.reshape(B, L, D + 1), pm.reshape(B, LM, D)


# speedup vs baseline: 1.8748x; 1.0280x over previous
import functools
import jax
import jax.numpy as jnp
from jax import lax
from jax.experimental import pallas as pl
from jax.experimental.pallas import tpu as pltpu
from jax.experimental.pallas import tpu_sc as plsc

_NW = 32
_CU = 512
_JU = _CU // 128
_CM = 256
_JM = _CM // 128


def _make_sc_kernel(n_u, n_m, D, pu_w, pm_w):
    mesh = plsc.VectorSubcoreMesh(core_axis_name="c", subcore_axis_name="s")

    @functools.partial(
        pl.kernel,
        out_type=(
            jax.ShapeDtypeStruct((n_u, D + 1), jnp.float32),
            jax.ShapeDtypeStruct((n_m, D), jnp.float32),
        ),
        mesh=mesh,
        compiler_params=pltpu.CompilerParams(
            use_tc_tiling_on_sc=False, needs_layout_passes=False),
        scratch_types=[
            pltpu.VMEM((_JU, 128), jnp.int32),
            pltpu.VMEM((_CU, D + 1), jnp.float32),
            pltpu.VMEM((_CM, D), jnp.float32),
            pltpu.VMEM((_CU,), jnp.float32),
            pltpu.SemaphoreType.DMA,
        ],
    )
    def k(aug_hbm, tab_hbm, iu_hbm, im_hbm, x_hbm, out_hbm, pm_hbm,
          idx_v, rows_v, pmr_v, xc_v, sem):
        wid = lax.axis_index("s") * 2 + lax.axis_index("c")

        def u_body(j, carry):
            base = wid * pu_w + j * _CU
            pltpu.sync_copy(iu_hbm.at[pl.ds(base // 128, _JU)], idx_v)
            cps = [
                pltpu.async_copy(
                    aug_hbm.at[idx_v.at[t]],
                    rows_v.at[pl.ds(t * 128, 128)], sem)
                for t in range(_JU)
            ]
            pltpu.sync_copy(x_hbm.at[pl.ds(base, _CU)], xc_v)
            for cp in cps:
                cp.wait()
            zero16 = jnp.zeros((16,), jnp.int32)
            for t in range(_CU // 16):
                vals = xc_v[pl.ds(t * 16, 16)]
                ridx = lax.iota(jnp.int32, 16) + t * 16
                plsc.store_scatter(rows_v, [ridx, zero16], vals)
            pltpu.sync_copy(rows_v, out_hbm.at[pl.ds(base, _CU)])
            return carry

        lax.fori_loop(0, pu_w // _CU, u_body, 0)

        def m_body(j, carry):
            base = wid * pm_w + j * _CM
            pltpu.sync_copy(im_hbm.at[pl.ds(base // 128, _JM)],
                            idx_v.at[pl.ds(0, _JM)])
            cps = [
                pltpu.async_copy(
                    tab_hbm.at[idx_v.at[t]],
                    pmr_v.at[pl.ds(t * 128, 128)], sem)
                for t in range(_JM)
            ]
            for cp in cps:
                cp.wait()
            pltpu.sync_copy(pmr_v, pm_hbm.at[pl.ds(base, _CM)])
            return carry

        lax.fori_loop(0, pm_w // _CM, m_body, 0)

    return k


def kernel(x, positions_unmask, positions_mask, pos_table):
    B, L = x.shape
    LM = positions_mask.shape[1]
    D = pos_table.shape[1]
    n_u = B * L
    n_m = B * LM
    pu_w = n_u // _NW
    pm_w = n_m // _NW

    xf = x.astype(jnp.float32).reshape(n_u)
    iu = positions_unmask.astype(jnp.int32).reshape(n_u // 128, 128)
    im = positions_mask.astype(jnp.int32).reshape(n_m // 128, 128)
    aug = jnp.concatenate(
        [jnp.zeros((pos_table.shape[0], 1), jnp.float32), pos_table], axis=1)

    k = _make_sc_kernel(n_u, n_m, D, pu_w, pm_w)
    out, pm = k(aug, pos_table, iu, im, xf)
    return out.reshape(B, L, D + 1), pm.reshape(B, LM, D)
